# Initial kernel scaffold; baseline (speedup 1.0000x reference)
import jax
import jax.numpy as jnp
from jax.experimental import pallas as pl


def kernel(x, indices, W, b):
    raise NotImplementedError("write your pallas kernel here")


# speedup vs baseline: 2.9182x; 2.9182x over previous
import functools

import jax
import jax.numpy as jnp
from jax import lax
from jax.experimental import pallas as pl
from jax.experimental.pallas import tpu as pltpu
from jax.experimental.pallas import tpu_sc as plsc

N_NODES = 50000
D = 128
SEQ = 9
OUT_CH = 128

NW = 32
CHUNK = 128
FLAT = N_NODES * SEQ
N_CHUNKS = -(-FLAT // (CHUNK * NW)) * NW
FLAT_PAD = N_CHUNKS * CHUNK
CHUNKS_PER_W = N_CHUNKS // NW

M_BLK = 400
N_MBLKS = N_NODES // M_BLK


def _sc_gather_body(x_hbm, idx_hbm, out_hbm, idx_v, rows_v, sem):
    wid = lax.axis_index("s") * 2 + lax.axis_index("c")
    base_c = wid * CHUNKS_PER_W
    pltpu.sync_copy(idx_hbm.at[wid], idx_v)

    def step(c, carry):
        pltpu.async_copy(x_hbm.at[idx_v.at[c]], rows_v, sem).wait()
        pltpu.sync_copy(rows_v, out_hbm.at[pl.ds((base_c + c) * CHUNK, CHUNK)])
        return carry

    lax.fori_loop(0, CHUNKS_PER_W, step, 0)


@jax.jit
def _sc_gather(x, idx2d):
    mesh = plsc.VectorSubcoreMesh(core_axis_name="c", subcore_axis_name="s")
    k = functools.partial(
        pl.kernel,
        mesh=mesh,
        out_type=jax.ShapeDtypeStruct((FLAT_PAD, D), jnp.float32),
        scratch_types=[
            pltpu.VMEM((CHUNKS_PER_W, CHUNK), jnp.int32),
            pltpu.VMEM((CHUNK, D), jnp.float32),
            pltpu.SemaphoreType.DMA,
        ],
    )(_sc_gather_body)
    return k(x, idx2d)


def _mm_body(*refs):
    g_refs, w_ref, b_ref, o_ref = refs[:SEQ], refs[SEQ], refs[SEQ + 1], refs[SEQ + 2]
    acc = jnp.broadcast_to(b_ref[...], (M_BLK, OUT_CH))
    for s in range(SEQ):
        acc = acc + jnp.dot(g_refs[s][...], w_ref[s],
                            preferred_element_type=jnp.float32)
    o_ref[...] = acc


@jax.jit
def _tc_matmul(gathered, w3, b2):
    in_specs = [
        pl.BlockSpec((M_BLK, D), functools.partial(lambda i, s: (s * N_MBLKS + i, 0), s=s))
        for s in range(SEQ)
    ]
    in_specs.append(pl.BlockSpec((SEQ, D, OUT_CH), lambda i: (0, 0, 0)))
    in_specs.append(pl.BlockSpec((1, OUT_CH), lambda i: (0, 0)))
    return pl.pallas_call(
        _mm_body,
        grid=(N_MBLKS,),
        in_specs=in_specs,
        out_specs=pl.BlockSpec((M_BLK, OUT_CH), lambda i: (i, 0)),
        out_shape=jax.ShapeDtypeStruct((N_NODES, OUT_CH), jnp.float32),
    )(*([gathered] * SEQ), w3, b2)


def kernel(x, indices, W, b):
    flat = indices.astype(jnp.int32).T.reshape(-1)
    flat = jnp.pad(flat, (0, FLAT_PAD - FLAT))
    idx2d = flat.reshape(NW, CHUNKS_PER_W, CHUNK)
    gathered = _sc_gather(x, idx2d)
    w3 = W.reshape(SEQ, D, OUT_CH)
    return _tc_matmul(gathered, w3, b.reshape(1, OUT_CH))


# speedup vs baseline: 3.2155x; 1.1019x over previous
import functools

import jax
import jax.numpy as jnp
from jax import lax
from jax.experimental import pallas as pl
from jax.experimental.pallas import tpu as pltpu
from jax.experimental.pallas import tpu_sc as plsc

N_NODES = 50000
D = 128
SEQ = 9
OUT_CH = 128

NW = 32
CHUNK = 128
FLAT = N_NODES * SEQ
N_CHUNKS = -(-FLAT // (CHUNK * NW)) * NW
FLAT_PAD = N_CHUNKS * CHUNK
CHUNKS_PER_W = N_CHUNKS // NW

M_BLK = 400
N_MBLKS = N_NODES // M_BLK


def _sc_gather_body(x_hbm, idx_hbm, out_hbm, idx_v, buf_a, buf_b, gs_a, gs_b, ws_a, ws_b):
    wid = lax.axis_index("s") * 2 + lax.axis_index("c")
    base_c = wid * CHUNKS_PER_W
    pltpu.sync_copy(idx_hbm.at[wid], idx_v)

    bufs, gsems, wsems = (buf_a, buf_b), (gs_a, gs_b), (ws_a, ws_b)

    def g_start(c, b):
        pltpu.async_copy(x_hbm.at[idx_v.at[c]], bufs[b], gsems[b])

    def g_wait(c, b):
        pltpu.make_async_copy(x_hbm.at[idx_v.at[c]], bufs[b], gsems[b]).wait()

    def out_slice(c):
        return out_hbm.at[pl.ds((base_c + c) * CHUNK, CHUNK)]

    def w_start(c, b):
        pltpu.async_copy(bufs[b], out_slice(c), wsems[b])

    def w_wait(c, b):
        pltpu.make_async_copy(bufs[b], out_slice(c), wsems[b]).wait()

    g_start(0, 0)
    g_wait(0, 0)
    w_start(0, 0)
    g_start(1, 1)

    def step(i, carry):
        c1 = 1 + 2 * i
        g_wait(c1, 1)
        w_start(c1, 1)
        w_wait(c1 - 1, 0)
        g_start(c1 + 1, 0)
        c2 = c1 + 1
        g_wait(c2, 0)
        w_start(c2, 0)
        w_wait(c2 - 1, 1)
        g_start(c2 + 1, 1)
        return carry

    lax.fori_loop(0, (CHUNKS_PER_W - 2) // 2, step, 0)
    c = CHUNKS_PER_W - 1
    g_wait(c, 1)
    w_start(c, 1)
    w_wait(c - 1, 0)
    w_wait(c, 1)


@jax.jit
def _sc_gather(x, idx2d):
    mesh = plsc.VectorSubcoreMesh(core_axis_name="c", subcore_axis_name="s")
    k = functools.partial(
        pl.kernel,
        mesh=mesh,
        out_type=jax.ShapeDtypeStruct((FLAT_PAD, D), jnp.float32),
        scratch_types=[
            pltpu.VMEM((CHUNKS_PER_W, CHUNK), jnp.int32),
            pltpu.VMEM((CHUNK, D), jnp.float32),
            pltpu.VMEM((CHUNK, D), jnp.float32),
            pltpu.SemaphoreType.DMA,
            pltpu.SemaphoreType.DMA,
            pltpu.SemaphoreType.DMA,
            pltpu.SemaphoreType.DMA,
        ],
    )(_sc_gather_body)
    return k(x, idx2d)


def _mm_body(*refs):
    g_refs, w_ref, b_ref, o_ref = refs[:SEQ], refs[SEQ], refs[SEQ + 1], refs[SEQ + 2]
    acc = jnp.broadcast_to(b_ref[...], (M_BLK, OUT_CH))
    for s in range(SEQ):
        acc = acc + jnp.dot(g_refs[s][...], w_ref[s],
                            preferred_element_type=jnp.float32)
    o_ref[...] = acc


@jax.jit
def _tc_matmul(gathered, w3, b2):
    in_specs = [
        pl.BlockSpec((M_BLK, D), functools.partial(lambda i, s: (s * N_MBLKS + i, 0), s=s))
        for s in range(SEQ)
    ]
    in_specs.append(pl.BlockSpec((SEQ, D, OUT_CH), lambda i: (0, 0, 0)))
    in_specs.append(pl.BlockSpec((1, OUT_CH), lambda i: (0, 0)))
    return pl.pallas_call(
        _mm_body,
        grid=(N_MBLKS,),
        in_specs=in_specs,
        out_specs=pl.BlockSpec((M_BLK, OUT_CH), lambda i: (i, 0)),
        out_shape=jax.ShapeDtypeStruct((N_NODES, OUT_CH), jnp.float32),
    )(*([gathered] * SEQ), w3, b2)


def kernel(x, indices, W, b):
    flat = indices.astype(jnp.int32).T.reshape(-1)
    flat = jnp.pad(flat, (0, FLAT_PAD - FLAT))
    idx2d = flat.reshape(NW, CHUNKS_PER_W, CHUNK)
    gathered = _sc_gather(x, idx2d)
    w3 = W.reshape(SEQ, D, OUT_CH)
    return _tc_matmul(gathered, w3, b.reshape(1, OUT_CH))


# speedup vs baseline: 3.5560x; 1.1059x over previous
import functools

import jax
import jax.numpy as jnp
from jax import lax
from jax.experimental import pallas as pl
from jax.experimental.pallas import tpu as pltpu
from jax.experimental.pallas import tpu_sc as plsc

N_NODES = 50000
D = 128
SEQ = 9
OUT_CH = 128

NW = 32
CHUNK = 128

GROUPS = 5
NODES_G = N_NODES // GROUPS
FLAT_G = NODES_G * SEQ
CPW = -(-FLAT_G // (CHUNK * NW))
FLAT_G_PAD = NW * CPW * CHUNK

M_BLK = 400
N_MBLKS = NODES_G // M_BLK


def _sc_gather_body(x_hbm, idx_hbm, out_hbm, idx_v, buf_a, buf_b, gs_a, gs_b, ws_a, ws_b):
    wid = lax.axis_index("s") * 2 + lax.axis_index("c")
    base_c = wid * CPW
    pltpu.sync_copy(idx_hbm.at[wid], idx_v)

    bufs, gsems, wsems = (buf_a, buf_b), (gs_a, gs_b), (ws_a, ws_b)

    def g_start(c, b):
        pltpu.async_copy(x_hbm.at[idx_v.at[c]], bufs[b], gsems[b])

    def g_wait(c, b):
        pltpu.make_async_copy(x_hbm.at[idx_v.at[c]], bufs[b], gsems[b]).wait()

    def out_slice(c):
        return out_hbm.at[pl.ds((base_c + c) * CHUNK, CHUNK)]

    def w_start(c, b):
        pltpu.async_copy(bufs[b], out_slice(c), wsems[b])

    def w_wait(c, b):
        pltpu.make_async_copy(bufs[b], out_slice(c), wsems[b]).wait()

    g_start(0, 0)
    g_wait(0, 0)
    w_start(0, 0)
    g_start(1, 1)

    def step(i, carry):
        c1 = 1 + 2 * i
        g_wait(c1, 1)
        w_start(c1, 1)
        w_wait(c1 - 1, 0)
        g_start(c1 + 1, 0)
        c2 = c1 + 1
        g_wait(c2, 0)
        w_start(c2, 0)
        w_wait(c2 - 1, 1)
        g_start(c2 + 1, 1)
        return carry

    lax.fori_loop(0, (CPW - 2) // 2, step, 0)
    c = CPW - 1
    g_wait(c, 1)
    w_start(c, 1)
    w_wait(c - 1, 0)
    w_wait(c, 1)


def _sc_gather(x, idx3d):
    mesh = plsc.VectorSubcoreMesh(core_axis_name="c", subcore_axis_name="s")
    k = functools.partial(
        pl.kernel,
        mesh=mesh,
        out_type=jax.ShapeDtypeStruct((FLAT_G_PAD, D), jnp.float32),
        scratch_types=[
            pltpu.VMEM((CPW, CHUNK), jnp.int32),
            pltpu.VMEM((CHUNK, D), jnp.float32),
            pltpu.VMEM((CHUNK, D), jnp.float32),
            pltpu.SemaphoreType.DMA,
            pltpu.SemaphoreType.DMA,
            pltpu.SemaphoreType.DMA,
            pltpu.SemaphoreType.DMA,
        ],
    )(_sc_gather_body)
    return k(x, idx3d)


def _mm_body(*refs):
    g_refs, w_ref, b_ref, o_ref = refs[:SEQ], refs[SEQ], refs[SEQ + 1], refs[SEQ + 2]
    acc = jnp.broadcast_to(b_ref[...], (M_BLK, OUT_CH))
    for s in range(SEQ):
        acc = acc + jnp.dot(g_refs[s][...], w_ref[s],
                            preferred_element_type=jnp.float32)
    o_ref[...] = acc


def _tc_matmul(gathered, w3, b2):
    in_specs = [
        pl.BlockSpec((M_BLK, D), functools.partial(lambda i, s: (s * N_MBLKS + i, 0), s=s))
        for s in range(SEQ)
    ]
    in_specs.append(pl.BlockSpec((SEQ, D, OUT_CH), lambda i: (0, 0, 0)))
    in_specs.append(pl.BlockSpec((1, OUT_CH), lambda i: (0, 0)))
    return pl.pallas_call(
        _mm_body,
        grid=(N_MBLKS,),
        in_specs=in_specs,
        out_specs=pl.BlockSpec((M_BLK, OUT_CH), lambda i: (i, 0)),
        out_shape=jax.ShapeDtypeStruct((NODES_G, OUT_CH), jnp.float32),
    )(*([gathered] * SEQ), w3, b2)


def kernel(x, indices, W, b):
    w3 = W.reshape(SEQ, D, OUT_CH)
    b2 = b.reshape(1, OUT_CH)
    outs = []
    for g in range(GROUPS):
        idx_g = indices[g * NODES_G:(g + 1) * NODES_G].astype(jnp.int32).T.reshape(-1)
        idx_g = jnp.pad(idx_g, (0, FLAT_G_PAD - FLAT_G)).reshape(NW, CPW, CHUNK)
        gathered = _sc_gather(x, idx_g)
        outs.append(_tc_matmul(gathered, w3, b2))
    return jnp.concatenate(outs, axis=0)
